# Initial kernel scaffold; baseline (speedup 1.0000x reference)
import jax
import jax.numpy as jnp
from jax.experimental import pallas as pl


def kernel(h, x, edge_attr, params, edges):
    raise NotImplementedError("write your pallas kernel here")


# speedup vs baseline: 2.2815x; 2.2815x over previous
import functools

import jax
import jax.numpy as jnp
from jax import lax
from jax.experimental import pallas as pl
from jax.experimental.pallas import tpu as pltpu
from jax.experimental.pallas import tpu_sc as plsc

NN = 10000
EE = 160000
HD = 256
C4 = 4

NC = 2
NS = 16
NW = NC * NS
EPW = EE // NW
CH = 200
NCH = EPW // CH
EPC = EE // NS
NCHM = EPC // CH

EBLK = 2000
NBLK = 2000


def _silu(v):
    return v * (1.0 / (1.0 + jnp.exp(-v)))


def _emb_in_body(h_ref, win_ref, bin_ref, wa_ref, be_ref, wb_ref,
                 hh_ref, p_ref, q_ref):
    hh = h_ref[...] @ win_ref[...] + bin_ref[...]
    hh_ref[...] = hh
    p_ref[...] = hh @ wa_ref[...] + be_ref[...]
    q_ref[...] = hh @ wb_ref[...]


def _edge_body(grow_ref, gcol_ref, crow_ref, ccol_ref, ea_ref,
               wea_ref, wr_ref, we1_ref, b1_ref, wc0_ref, bc_ref, c1_ref,
               m2a_ref, m2b_ref, t4_ref):
    d = crow_ref[...] - ccol_ref[...]
    radial = jnp.sum(d * d, axis=1, keepdims=True)
    t = grow_ref[...] + gcol_ref[...] + radial * wr_ref[...] \
        + ea_ref[...] @ wea_ref[...]
    m1 = _silu(t)
    m2 = _silu(m1 @ we1_ref[...] + b1_ref[...])
    cm = _silu(m2 @ wc0_ref[...] + bc_ref[...])
    s = jnp.sum(cm * c1_ref[...], axis=1, keepdims=True)
    tr = d * s
    lane = lax.broadcasted_iota(jnp.int32, tr.shape, 1)
    t4_ref[...] = jnp.where(lane == 3, 1.0, tr)
    m2a_ref[...] = m2[:, :128]
    m2b_ref[...] = m2[:, 128:]


def _tred_body(tp_ref, out_ref):
    out_ref[...] = jnp.sum(tp_ref[...], axis=0, keepdims=True)


def _tred_call(tpart):
    return pl.pallas_call(
        _tred_body,
        out_shape=jax.ShapeDtypeStruct((1, NN * C4), jnp.float32),
    )(tpart).reshape(NN, C4)


def _coord_update(cp, sv):
    lane = lax.broadcasted_iota(jnp.int32, sv.shape, 1)
    cnt = jnp.sum(jnp.where(lane == 3, sv, 0.0), axis=1, keepdims=True)
    cnt = jnp.maximum(cnt, 1.0)
    return cp + jnp.where(lane < 3, sv, 0.0) / cnt


def _node_body(hh_ref, agga_ref, aggb_ref, sv_ref, cp_ref,
               wn0h_ref, wn0a_ref, wn0b_ref, bn0_ref, wn1_ref, bn1_ref,
               wa_ref, be_ref, wb_ref,
               hh2_ref, cp2_ref, p_ref, q_ref):
    u = _silu(hh_ref[...] @ wn0h_ref[...] + agga_ref[...] @ wn0a_ref[...]
              + aggb_ref[...] @ wn0b_ref[...] + bn0_ref[...])
    hh2 = hh_ref[...] + (u @ wn1_ref[...] + bn1_ref[...])
    hh2_ref[...] = hh2
    cp2_ref[...] = _coord_update(cp_ref[...], sv_ref[...])
    p_ref[...] = hh2 @ wa_ref[...] + be_ref[...]
    q_ref[...] = hh2 @ wb_ref[...]


def _final_body(hh_ref, agga_ref, aggb_ref, sv_ref, cp_ref,
                wn0h_ref, wn0a_ref, wn0b_ref, bn0_ref, wn1_ref, bn1_ref,
                wout_ref, bout_ref,
                hout_ref, cp2_ref):
    u = _silu(hh_ref[...] @ wn0h_ref[...] + agga_ref[...] @ wn0a_ref[...]
              + aggb_ref[...] @ wn0b_ref[...] + bn0_ref[...])
    hh2 = hh_ref[...] + (u @ wn1_ref[...] + bn1_ref[...])
    hout_ref[...] = hh2 @ wout_ref[...] + bout_ref[...]
    cp2_ref[...] = _coord_update(cp_ref[...], sv_ref[...])


def _row_spec(blk, width):
    return pl.BlockSpec((blk, width), lambda i: (i, 0))


def _full_spec(r, c):
    return pl.BlockSpec((r, c), lambda i: (0, 0))


def _emb_in_call(h, win, binr, wa, be, wb):
    return pl.pallas_call(
        _emb_in_body,
        grid=(NN // NBLK,),
        in_specs=[_row_spec(NBLK, HD), _full_spec(HD, HD), _full_spec(1, HD),
                  _full_spec(HD, HD), _full_spec(1, HD), _full_spec(HD, HD)],
        out_specs=[_row_spec(NBLK, HD)] * 3,
        out_shape=[jax.ShapeDtypeStruct((NN, HD), jnp.float32)] * 3,
    )(h, win, binr, wa, be, wb)


def _edge_call(grow, gcol, crow, ccol, ea, wea, wr, we1, b1, wc0, bc, c1r):
    return pl.pallas_call(
        _edge_body,
        grid=(EE // EBLK,),
        in_specs=[_row_spec(EBLK, HD), _row_spec(EBLK, HD),
                  _row_spec(EBLK, C4), _row_spec(EBLK, C4),
                  _row_spec(EBLK, 16),
                  _full_spec(16, HD), _full_spec(1, HD),
                  _full_spec(HD, HD), _full_spec(1, HD),
                  _full_spec(HD, HD), _full_spec(1, HD), _full_spec(1, HD)],
        out_specs=[_row_spec(EBLK, 128), _row_spec(EBLK, 128),
                   _row_spec(EBLK, C4)],
        out_shape=[jax.ShapeDtypeStruct((EE, 128), jnp.float32),
                   jax.ShapeDtypeStruct((EE, 128), jnp.float32),
                   jax.ShapeDtypeStruct((EE, C4), jnp.float32)],
    )(grow, gcol, crow, ccol, ea, wea, wr, we1, b1, wc0, bc, c1r)


def _node_call(hh, agga, aggb, svp, cp, wn0h, wn0a, wn0b, bn0, wn1, bn1,
               wa, be, wb):
    return pl.pallas_call(
        _node_body,
        grid=(NN // NBLK,),
        in_specs=[_row_spec(NBLK, HD), _row_spec(NBLK, 128),
                  _row_spec(NBLK, 128), _row_spec(NBLK, C4), _row_spec(NBLK, C4),
                  _full_spec(HD, HD), _full_spec(128, HD), _full_spec(128, HD),
                  _full_spec(1, HD), _full_spec(HD, HD), _full_spec(1, HD),
                  _full_spec(HD, HD), _full_spec(1, HD), _full_spec(HD, HD)],
        out_specs=[_row_spec(NBLK, HD), _row_spec(NBLK, C4),
                   _row_spec(NBLK, HD), _row_spec(NBLK, HD)],
        out_shape=[jax.ShapeDtypeStruct((NN, HD), jnp.float32),
                   jax.ShapeDtypeStruct((NN, C4), jnp.float32),
                   jax.ShapeDtypeStruct((NN, HD), jnp.float32),
                   jax.ShapeDtypeStruct((NN, HD), jnp.float32)],
    )(hh, agga, aggb, svp, cp, wn0h, wn0a, wn0b, bn0, wn1, bn1, wa, be, wb)


def _final_call(hh, agga, aggb, svp, cp, wn0h, wn0a, wn0b, bn0, wn1, bn1,
                wout, bout):
    return pl.pallas_call(
        _final_body,
        grid=(NN // NBLK,),
        in_specs=[_row_spec(NBLK, HD), _row_spec(NBLK, 128),
                  _row_spec(NBLK, 128), _row_spec(NBLK, C4), _row_spec(NBLK, C4),
                  _full_spec(HD, HD), _full_spec(128, HD), _full_spec(128, HD),
                  _full_spec(1, HD), _full_spec(HD, HD), _full_spec(1, HD),
                  _full_spec(HD, HD), _full_spec(1, HD)],
        out_specs=[_row_spec(NBLK, HD), _row_spec(NBLK, C4)],
        out_shape=[jax.ShapeDtypeStruct((NN, HD), jnp.float32),
                   jax.ShapeDtypeStruct((NN, C4), jnp.float32)],
    )(hh, agga, aggb, svp, cp, wn0h, wn0a, wn0b, bn0, wn1, bn1, wout, bout)


@functools.cache
def _build_sc_gather():
    mesh = plsc.VectorSubcoreMesh(core_axis_name="c", subcore_axis_name="s")

    @functools.partial(
        pl.kernel,
        mesh=mesh,
        out_type=[jax.ShapeDtypeStruct((EE, HD), jnp.float32),
                  jax.ShapeDtypeStruct((EE, HD), jnp.float32),
                  jax.ShapeDtypeStruct((EE * C4,), jnp.float32),
                  jax.ShapeDtypeStruct((EE * C4,), jnp.float32)],
        scratch_types=[pltpu.VMEM((CH,), jnp.int32),
                       pltpu.VMEM((CH, HD), jnp.float32),
                       pltpu.VMEM((CH * C4,), jnp.float32),
                       pltpu.VMEM((NN * C4,), jnp.float32),
                       pltpu.SemaphoreType.DMA],
        compiler_params=pltpu.CompilerParams(needs_layout_passes=False),
    )
    def gather_k(p_hbm, q_hbm, cf_hbm, row_hbm, col_hbm,
                 grow_hbm, gcol_hbm, crow_hbm, ccol_hbm,
                 idx_v, rows_v, obuf_v, ctab_v, sem):
        wid = lax.axis_index("s") * NC + lax.axis_index("c")
        base = wid * EPW
        pltpu.sync_copy(cf_hbm, ctab_v)
        lanes = lax.iota(jnp.int32, 16)
        sub = lanes >> 2
        comp = lanes & 3

        def side(tab_hbm, idxsrc_hbm, gout_hbm, cout_hbm, i):
            off = base + i * CH
            sl = pl.ds(off, CH)
            pltpu.sync_copy(idxsrc_hbm.at[sl], idx_v)
            cp = pltpu.async_copy(tab_hbm.at[idx_v], rows_v, sem)

            def vreg(v, _):
                rows = plsc.load_gather(idx_v, [v * 4 + sub])
                vals = plsc.load_gather(ctab_v, [rows * C4 + comp])
                obuf_v[pl.ds(v * 16, 16)] = vals
                return ()

            lax.fori_loop(0, CH * C4 // 16, vreg, ())
            pltpu.sync_copy(obuf_v, cout_hbm.at[pl.ds(off * C4, CH * C4)])
            cp.wait()
            pltpu.sync_copy(rows_v, gout_hbm.at[sl])

        def chunk(i, _):
            side(p_hbm, row_hbm, grow_hbm, crow_hbm, i)
            side(q_hbm, col_hbm, gcol_hbm, ccol_hbm, i)
            return ()

        lax.fori_loop(0, NCH, chunk, ())

    return gather_k


def _sc_gather(p, q, cf, row, col):
    return _build_sc_gather()(p, q, cf, row, col)


@functools.cache
def _build_sc_scatter_m2():
    mesh = plsc.VectorSubcoreMesh(core_axis_name="c", subcore_axis_name="s")

    @functools.partial(
        pl.kernel,
        mesh=mesh,
        out_type=[jax.ShapeDtypeStruct((NN, 128), jnp.float32),
                  jax.ShapeDtypeStruct((NN, 128), jnp.float32)],
        scratch_types=[pltpu.VMEM((CH,), jnp.int32),
                       pltpu.VMEM((CH, 128), jnp.float32),
                       pltpu.VMEM_SHARED((NN, 128), jnp.float32)],
        compiler_params=pltpu.CompilerParams(needs_layout_passes=False),
    )
    def scatter_k(m2a_hbm, m2b_hbm, row_hbm, zb_hbm,
                  agga_hbm, aggb_hbm,
                  idxm_v, buf_v, acc_sh):
        c = lax.axis_index("c")
        s = lax.axis_index("s")

        @pl.when(s == 0)
        def _():
            pltpu.sync_copy(zb_hbm, acc_sh)

        plsc.subcore_barrier()
        mbase = s * EPC

        def chunk(i, _):
            msl = pl.ds(mbase + i * CH, CH)
            pltpu.sync_copy(row_hbm.at[msl], idxm_v)

            @pl.when(c == 0)
            def _():
                pltpu.sync_copy(m2a_hbm.at[msl], buf_v)

            @pl.when(c == 1)
            def _():
                pltpu.sync_copy(m2b_hbm.at[msl], buf_v)

            pltpu.sync_copy(buf_v, acc_sh.at[idxm_v], add=True)
            return ()

        lax.fori_loop(0, NCHM, chunk, ())
        plsc.subcore_barrier()

        @pl.when(s < 10)
        def _():
            sl = pl.ds(s * 1000, 1000)

            @pl.when(c == 0)
            def _():
                pltpu.sync_copy(acc_sh.at[sl], agga_hbm.at[sl])

            @pl.when(c == 1)
            def _():
                pltpu.sync_copy(acc_sh.at[sl], aggb_hbm.at[sl])

    return scatter_k


CHT = 1000
NCHT = EPW // CHT


@functools.cache
def _build_sc_scatter_t4():
    mesh = plsc.VectorSubcoreMesh(core_axis_name="c", subcore_axis_name="s")

    @functools.partial(
        pl.kernel,
        mesh=mesh,
        out_type=jax.ShapeDtypeStruct((NW, NN * C4), jnp.float32),
        scratch_types=[pltpu.VMEM((CHT,), jnp.int32),
                       pltpu.VMEM((CHT * C4,), jnp.float32),
                       pltpu.VMEM((NN * C4,), jnp.float32)],
        compiler_params=pltpu.CompilerParams(needs_layout_passes=False),
    )
    def scatter_t4_k(t4_hbm, row_hbm, tpart_hbm, idxt_v, tbuf_v, tacc_v):
        c = lax.axis_index("c")
        s = lax.axis_index("s")
        wid = s * NC + c
        lanes = lax.iota(jnp.int32, 16)
        sub = lanes >> 2
        comp = lanes & 3
        zeros16 = jnp.zeros((16,), jnp.float32)

        def zt(v, _):
            tacc_v[pl.ds(v * 16, 16)] = zeros16
            return ()

        lax.fori_loop(0, NN * C4 // 16, zt, ())
        tbase = wid * EPW

        def chunk(i, _):
            toff = tbase + i * CHT
            pltpu.sync_copy(row_hbm.at[pl.ds(toff, CHT)], idxt_v)
            pltpu.sync_copy(t4_hbm.at[pl.ds(toff * C4, CHT * C4)], tbuf_v)

            def vreg(v, _):
                vals = tbuf_v[pl.ds(v * 16, 16)]
                rows = plsc.load_gather(idxt_v, [v * 4 + sub])
                plsc.addupdate_scatter(tacc_v, [rows * C4 + comp], vals)
                return ()

            lax.fori_loop(0, CHT * C4 // 16, vreg, ())
            return ()

        lax.fori_loop(0, NCHT, chunk, ())
        pltpu.sync_copy(tacc_v, tpart_hbm.at[wid])

    return scatter_t4_k


def _sc_scatter(m2a, m2b, t4, row, zb):
    agga, aggb = _build_sc_scatter_m2()(m2a, m2b, row, zb)
    tpart = _build_sc_scatter_t4()(t4, row)
    return agga, aggb, tpart


def _split_edge0(lp):
    we0, be0 = lp['edge0']
    wa = we0[:HD]
    wb = we0[HD:2 * HD]
    wr = we0[2 * HD:2 * HD + 1]
    wea = we0[2 * HD + 1:]
    return wa, wb, wr, wea, be0.reshape(1, HD)


def kernel(h, x, edge_attr, params, edges):
    row = edges[0]
    col = edges[1]
    coordp = jnp.concatenate(
        [x, jnp.zeros((NN, C4 - 3), jnp.float32)], axis=1)
    zb = jnp.zeros((NN, 128), jnp.float32)

    layers = params['layers']
    wa0, wb0, _, _, be0 = _split_edge0(layers[0])
    win, bin_ = params['emb_in']
    hh, p, q = _emb_in_call(h, win, bin_.reshape(1, HD), wa0, be0, wb0)

    hout = None
    for li in range(4):
        lp = layers[li]
        _, _, wr, wea, _ = _split_edge0(lp)
        we1, b1 = lp['edge1']
        wc0, bc = lp['coord0']
        c1 = lp['coord1']

        grow, gcol, crowf, ccolf = _sc_gather(
            p, q, coordp.reshape(-1), row, col)
        m2a, m2b, t4 = _edge_call(
            grow, gcol, crowf.reshape(EE, C4), ccolf.reshape(EE, C4),
            edge_attr, wea, wr, we1, b1.reshape(1, HD), wc0,
            bc.reshape(1, HD), c1.reshape(1, HD))
        agga, aggb, tpart = _sc_scatter(m2a, m2b, t4.reshape(-1), row, zb)
        svec = _tred_call(tpart)

        wn0, bn0 = lp['node0']
        wn1, bn1 = lp['node1']
        wn0h = wn0[:HD]
        wn0a = wn0[HD:HD + 128]
        wn0b = wn0[HD + 128:]

        if li < 3:
            wa, wb, _, _, be = _split_edge0(layers[li + 1])
            hh, coordp, p, q = _node_call(
                hh, agga, aggb, svec, coordp,
                wn0h, wn0a, wn0b, bn0.reshape(1, HD), wn1,
                bn1.reshape(1, HD), wa, be, wb)
        else:
            wout, bout = params['emb_out']
            hout, coordp = _final_call(
                hh, agga, aggb, svec, coordp,
                wn0h, wn0a, wn0b, bn0.reshape(1, HD), wn1,
                bn1.reshape(1, HD), wout, bout.reshape(1, HD))

    return (hout, coordp[:, :3])
